# Initial kernel scaffold; baseline (speedup 1.0000x reference)
import jax
import jax.numpy as jnp
from jax.experimental import pallas as pl


def kernel(token_ids, lang_ids, token_table, lang_table, W_lang):
    raise NotImplementedError("write your pallas kernel here")


# speedup vs baseline: 2.0876x; 2.0876x over previous
import functools
import math

import numpy as np
import jax
import jax.numpy as jnp
from jax import lax
from jax.experimental import pallas as pl
from jax.experimental.pallas import tpu as pltpu
from jax.experimental.pallas import tpu_sc as plsc

_NC = 2
_NS = 16
_NW = _NC * _NS
_LANES = 16

_PB = 128
_CH = 128


def _build_pe(max_len: int, d_model: int) -> np.ndarray:
    position = np.arange(max_len)[:, None].astype(np.float32)
    div_term = np.exp(
        np.arange(0, d_model, 2).astype(np.float32) * (-math.log(10000.0) / d_model)
    )
    pe = np.zeros((max_len, d_model), dtype=np.float32)
    pe[:, 0::2] = np.sin(position * div_term)
    pe[:, 1::2] = np.cos(position * div_term)
    return pe


def _combo_body(lang_ref, w_ref, pe_ref, out_ref):
    lp = lax.dot_general(
        lang_ref[...], w_ref[...], (((1,), (1,)), ((), ())),
        preferred_element_type=jnp.float32,
    )
    out_ref[...] = pe_ref[...][:, None, :] + lp[None, :, :]


def _make_sc_embed(B, L, D, NL, scale):
    BW = B // _NW
    LB = L // _PB
    mesh = plsc.VectorSubcoreMesh(core_axis_name="c", subcore_axis_name="s")

    @functools.partial(
        pl.kernel,
        mesh=mesh,
        out_type=jax.ShapeDtypeStruct((B * L, D), jnp.float32),
        scratch_types=[
            pltpu.VMEM((_CH,), jnp.int32),
            pltpu.VMEM((_CH,), jnp.int32),
            pltpu.VMEM((_PB, NL, D), jnp.float32),
            pltpu.VMEM((_CH, D), jnp.float32),
            pltpu.SemaphoreType.DMA,
        ],
    )
    def _sc_embed(tok_hbm, lang_hbm, table_hbm, combo_hbm, out_hbm,
                  idx_v, langv, combo_v, rows_v, sem):
        wid = lax.axis_index("s") * _NC + lax.axis_index("c")

        for lb in range(LB):
            pltpu.sync_copy(combo_hbm.at[pl.ds(lb * _PB, _PB)], combo_v)

            def batch_body(bi, _, lb=lb):
                off = (wid * BW + bi) * L + lb * _PB
                pltpu.sync_copy(tok_hbm.at[pl.ds(off, _CH)], idx_v)
                pltpu.sync_copy(lang_hbm.at[pl.ds(off, _CH)], langv)
                pltpu.async_copy(table_hbm.at[idx_v], rows_v, sem).wait()

                def group_body(g, _):
                    lidv = langv[pl.ds(g * _LANES, _LANES)]
                    for k in range(_LANES):
                        lid = lidv[k]
                        t = g * _LANES + k
                        for c in range(D // _LANES):
                            sl = pl.ds(c * _LANES, _LANES)
                            rows_v[t, sl] = (
                                rows_v[t, sl] * scale + combo_v[t, lid, sl]
                            )
                    return 0

                lax.fori_loop(0, _CH // _LANES, group_body, 0)
                pltpu.sync_copy(rows_v, out_hbm.at[pl.ds(off, _CH), :])
                return 0

            lax.fori_loop(0, BW, batch_body, 0)

    return _sc_embed


def kernel(token_ids, lang_ids, token_table, lang_table, W_lang):
    B, L = token_ids.shape
    V, D = token_table.shape
    NL, LD = lang_table.shape
    scale = math.sqrt(D)

    pe = jnp.asarray(_build_pe(L, D))

    combo = pl.pallas_call(
        _combo_body,
        out_shape=jax.ShapeDtypeStruct((L, NL, D), jnp.float32),
    )(lang_table, W_lang, pe)

    sc_embed = _make_sc_embed(B, L, D, NL, scale)
    out_flat = sc_embed(
        token_ids.reshape(-1), lang_ids.reshape(-1), token_table, combo
    )
    return out_flat.reshape(B, L, D)
